# Initial kernel scaffold; baseline (speedup 1.0000x reference)
import jax
import jax.numpy as jnp
from jax.experimental import pallas as pl


def kernel(x, edge_index, W1, b1, W2, b2, W3, b3, W4, b4, W5, b5):
    raise NotImplementedError("write your pallas kernel here")


# speedup vs baseline: 6.3526x; 6.3526x over previous
import functools

import jax
import jax.numpy as jnp
from jax import lax
from jax.experimental import pallas as pl
from jax.experimental.pallas import tpu as pltpu
from jax.experimental.pallas import tpu_sc as plsc

N = 10000
E = 320000
D = 128

NC = 2
NS = 16
NW = NC * NS

CH = 128
NCH = -(-E // (NW * CH))
NCH += NCH % 2
EP = NW * NCH * CH
NPAD = -(-(N + 1) // (16 * NS)) * (16 * NS)
RPT = NPAD // NS

_MESH = plsc.VectorSubcoreMesh(core_axis_name="c", subcore_axis_name="s")


@functools.partial(
    pl.kernel,
    out_type=jax.ShapeDtypeStruct((NC, NPAD, D), jnp.float32),
    mesh=_MESH,
    scratch_types=[
        pltpu.VMEM((CH, D), jnp.float32),
        pltpu.VMEM((NCH, CH), jnp.int32),
        pltpu.VMEM_SHARED((NPAD, D), jnp.float32),
    ],
)
def _sc_degree(ones_hbm, dst_hbm, z16_hbm, out_hbm, ones_v, dst_v, acc_sh):
    c = lax.axis_index("c")
    s = lax.axis_index("s")
    wid = c * NS + s
    pltpu.sync_copy(z16_hbm.at[pl.ds(s * RPT, RPT)], acc_sh.at[pl.ds(s * RPT, RPT)])
    pltpu.sync_copy(ones_hbm, ones_v)
    pltpu.sync_copy(dst_hbm.at[wid], dst_v)
    plsc.subcore_barrier()

    @pl.loop(0, NCH)
    def _(j):
        pltpu.sync_copy(ones_v, acc_sh.at[dst_v.at[j]], add=True)

    plsc.subcore_barrier()
    pltpu.sync_copy(acc_sh.at[pl.ds(s * RPT, RPT)], out_hbm.at[c, pl.ds(s * RPT, RPT)])


@functools.partial(
    pl.kernel,
    out_type=jax.ShapeDtypeStruct((NC, NPAD, D), jnp.float32),
    mesh=_MESH,
    scratch_types=[
        pltpu.VMEM((NCH, CH), jnp.int32),
        pltpu.VMEM((CH,), jnp.int32),
        pltpu.VMEM((CH,), jnp.int32),
        pltpu.VMEM((CH, D), jnp.float32),
        pltpu.VMEM((CH, D), jnp.float32),
        pltpu.SemaphoreType.DMA,
        pltpu.SemaphoreType.DMA,
        pltpu.VMEM_SHARED((NPAD, D), jnp.float32),
    ],
)
def _sc_scatter(hs_hbm, src_hbm, dst_hbm, z_hbm, out_hbm,
                dst_v, sidx0, sidx1, rows0, rows1, sem0, sem1, acc_sh):
    c = lax.axis_index("c")
    s = lax.axis_index("s")
    wid = c * NS + s
    pltpu.sync_copy(z_hbm.at[pl.ds(s * RPT, RPT)], acc_sh.at[pl.ds(s * RPT, RPT)])
    pltpu.sync_copy(dst_hbm.at[wid], dst_v)
    plsc.subcore_barrier()

    pltpu.sync_copy(src_hbm.at[wid, 0], sidx0)
    pltpu.async_copy(hs_hbm.at[sidx0], rows0, sem0)

    @pl.loop(0, NCH, step=2)
    def _(j):
        pltpu.sync_copy(src_hbm.at[wid, j + 1], sidx1)
        pltpu.async_copy(hs_hbm.at[sidx1], rows1, sem1)
        pltpu.make_async_copy(hs_hbm.at[sidx0], rows0, sem0).wait()
        pltpu.sync_copy(rows0, acc_sh.at[dst_v.at[j]], add=True)

        @pl.when(j + 2 < NCH)
        def _():
            pltpu.sync_copy(src_hbm.at[wid, j + 2], sidx0)
            pltpu.async_copy(hs_hbm.at[sidx0], rows0, sem0)

        pltpu.make_async_copy(hs_hbm.at[sidx1], rows1, sem1).wait()
        pltpu.sync_copy(rows1, acc_sh.at[dst_v.at[j + 1]], add=True)

    plsc.subcore_barrier()
    pltpu.sync_copy(acc_sh.at[pl.ds(s * RPT, RPT)], out_hbm.at[c, pl.ds(s * RPT, RPT)])


_R = 1000
_DOT = functools.partial(jnp.dot, preferred_element_type=jnp.float32,
                         precision=lax.Precision.HIGHEST)


def _tc_prep_body(x_ref, w_ref, degp_ref, hs_ref, dinv_ref):
    deg = degp_ref[0, :, 0:1] + degp_ref[1, :, 0:1] + 1.0
    dinv = lax.rsqrt(deg)
    hs_ref[...] = _DOT(x_ref[...], w_ref[...]) * dinv
    dinv_ref[...] = jnp.broadcast_to(dinv, (_R, D))


def _tc_mid_body(p_ref, hsp_ref, dinv_ref, b_ref, w_ref, out_ref):
    agg = p_ref[0] + p_ref[1] + hsp_ref[...]
    f = jnp.maximum(dinv_ref[...] * agg + b_ref[...], 0.0)
    out_ref[...] = _DOT(f, w_ref[...]) * dinv_ref[...]


def _tc_final_body(p_ref, hsp_ref, dinv_ref, b_ref, out_ref):
    agg = p_ref[0] + p_ref[1] + hsp_ref[...]
    out_ref[...] = dinv_ref[...] * agg + b_ref[...]


def _row_spec(width):
    return pl.BlockSpec((_R, width), lambda i: (i, 0))


def _part_spec(width):
    return pl.BlockSpec((NC, _R, width), lambda i: (0, i, 0))


def _full_spec(shape):
    return pl.BlockSpec(shape, lambda i: (0,) * len(shape))


_GRID = N // _R

_tc_prep = pl.pallas_call(
    _tc_prep_body,
    grid=(_GRID,),
    in_specs=[_row_spec(D), _full_spec((D, D)), _part_spec(D)],
    out_specs=[_row_spec(D), _row_spec(D)],
    out_shape=[jax.ShapeDtypeStruct((N, D), jnp.float32),
               jax.ShapeDtypeStruct((N, D), jnp.float32)],
)

_tc_mid = pl.pallas_call(
    _tc_mid_body,
    grid=(_GRID,),
    in_specs=[_part_spec(D), _row_spec(D), _row_spec(D),
              _full_spec((1, D)), _full_spec((D, D))],
    out_specs=_row_spec(D),
    out_shape=jax.ShapeDtypeStruct((N, D), jnp.float32),
)

_tc_final = pl.pallas_call(
    _tc_final_body,
    grid=(_GRID,),
    in_specs=[_part_spec(D), _row_spec(D), _row_spec(D), _full_spec((1, D))],
    out_specs=_row_spec(D),
    out_shape=jax.ShapeDtypeStruct((N, D), jnp.float32),
)


@jax.jit
def kernel(x, edge_index, W1, b1, W2, b2, W3, b3, W4, b4, W5, b5):
    src = edge_index[0]
    dst = edge_index[1]
    pad = EP - E
    src3 = jnp.concatenate([src, jnp.zeros((pad,), jnp.int32)]).reshape(NW, NCH, CH)
    dst3 = jnp.concatenate([dst, jnp.full((pad,), N, jnp.int32)]).reshape(NW, NCH, CH)

    zeros_nd = jnp.zeros((NPAD, D), jnp.float32)
    ones_ch = jnp.ones((CH, D), jnp.float32)

    degp = _sc_degree(ones_ch, dst3, zeros_nd)
    hs, dinv = _tc_prep(x, W1, degp)

    W5p = jnp.zeros((D, D), jnp.float32).at[:, :50].set(W5)
    b5p = jnp.zeros((1, D), jnp.float32).at[0, :50].set(b5)

    for b, W in ((b1, W2), (b2, W3), (b3, W4), (b4, W5p)):
        p = _sc_scatter(hs, src3, dst3, zeros_nd)
        hs = _tc_mid(p, hs, dinv, b.reshape(1, D) if b.ndim == 1 else b, W)

    p = _sc_scatter(hs, src3, dst3, zeros_nd)
    out = _tc_final(p, hs, dinv, b5p)
    return out[:, :50]


# speedup vs baseline: 11.9482x; 1.8808x over previous
import functools

import jax
import jax.numpy as jnp
from jax import lax
from jax.experimental import pallas as pl
from jax.experimental.pallas import tpu as pltpu
from jax.experimental.pallas import tpu_sc as plsc

N = 10000
E = 320000
D = 128

NC = 2
NS = 16
NW = NC * NS

CH = 128
NCH0 = 118
NCH1 = 40
E0 = NS * NCH0 * CH
CAP1 = NS * NCH1 * CH
assert E0 + CAP1 >= E
NPAD = 10112
RPT = NPAD // NS

_MESH = plsc.VectorSubcoreMesh(core_axis_name="c", subcore_axis_name="s")


@functools.partial(
    pl.kernel,
    out_type=jax.ShapeDtypeStruct((NC, NPAD, D), jnp.float32),
    mesh=_MESH,
    scratch_types=[
        pltpu.VMEM((CH, D), jnp.float32),
        pltpu.VMEM((NCH0, CH), jnp.int32),
        pltpu.VMEM_SHARED((NPAD, D), jnp.float32),
    ],
)
def _sc_degree(ones_hbm, dst_hbm, z_hbm, out_hbm, ones_v, dst_v, acc_sh):
    c = lax.axis_index("c")
    s = lax.axis_index("s")
    wid = c * NS + s
    pltpu.sync_copy(z_hbm.at[pl.ds(s * RPT, RPT)], acc_sh.at[pl.ds(s * RPT, RPT)])
    pltpu.sync_copy(ones_hbm, ones_v)
    plsc.subcore_barrier()

    def count(nch):
        pltpu.sync_copy(dst_hbm.at[wid, pl.ds(0, nch)], dst_v.at[pl.ds(0, nch)])

        @pl.loop(0, nch)
        def _(j):
            pltpu.sync_copy(ones_v, acc_sh.at[dst_v.at[j]], add=True)

    @pl.when(c == 0)
    def _():
        count(NCH0)

    @pl.when(c == 1)
    def _():
        count(NCH1)

    plsc.subcore_barrier()
    pltpu.sync_copy(acc_sh.at[pl.ds(s * RPT, RPT)], out_hbm.at[c, pl.ds(s * RPT, RPT)])


@functools.partial(
    pl.kernel,
    out_type=jax.ShapeDtypeStruct((NC, NPAD, D), jnp.float32),
    mesh=_MESH,
    scratch_types=[
        pltpu.VMEM((NCH0, CH), jnp.int32),
        pltpu.VMEM((CH,), jnp.int32),
        pltpu.VMEM((CH,), jnp.int32),
        pltpu.VMEM((CH, D), jnp.float32),
        pltpu.VMEM((CH, D), jnp.float32),
        pltpu.SemaphoreType.DMA,
        pltpu.SemaphoreType.DMA,
        pltpu.VMEM_SHARED((NPAD, D), jnp.float32),
    ],
)
def _sc_scatter(hs_hbm, src_hbm, dst_hbm, z_hbm, out_hbm,
                dst_v, sidx0, sidx1, rows0, rows1, sem0, sem1, acc_sh):
    c = lax.axis_index("c")
    s = lax.axis_index("s")
    wid = c * NS + s
    pltpu.sync_copy(z_hbm.at[pl.ds(s * RPT, RPT)], acc_sh.at[pl.ds(s * RPT, RPT)])
    plsc.subcore_barrier()

    def run(nch):
        pltpu.sync_copy(dst_hbm.at[wid, pl.ds(0, nch)], dst_v.at[pl.ds(0, nch)])
        pltpu.sync_copy(src_hbm.at[wid, 0], sidx0)
        pltpu.async_copy(hs_hbm.at[sidx0], rows0, sem0)

        @pl.loop(0, nch, step=2)
        def _(j):
            pltpu.sync_copy(src_hbm.at[wid, j + 1], sidx1)
            pltpu.async_copy(hs_hbm.at[sidx1], rows1, sem1)
            pltpu.make_async_copy(hs_hbm.at[sidx0], rows0, sem0).wait()
            pltpu.sync_copy(rows0, acc_sh.at[dst_v.at[j]], add=True)

            @pl.when(j + 2 < nch)
            def _():
                pltpu.sync_copy(src_hbm.at[wid, j + 2], sidx0)
                pltpu.async_copy(hs_hbm.at[sidx0], rows0, sem0)

            pltpu.make_async_copy(hs_hbm.at[sidx1], rows1, sem1).wait()
            pltpu.sync_copy(rows1, acc_sh.at[dst_v.at[j + 1]], add=True)

    @pl.when(c == 0)
    def _():
        run(NCH0)

    @pl.when(c == 1)
    def _():
        run(NCH1)

    plsc.subcore_barrier()
    pltpu.sync_copy(acc_sh.at[pl.ds(s * RPT, RPT)], out_hbm.at[c, pl.ds(s * RPT, RPT)])


_R = 1000
_DOT = functools.partial(jnp.dot, preferred_element_type=jnp.float32,
                         precision=lax.Precision.HIGHEST)


def _tc_prep_body(x_ref, w_ref, degp_ref, hs_ref, dinv_ref):
    deg = degp_ref[0, :, 0:1] + degp_ref[1, :, 0:1] + 1.0
    dinv = lax.rsqrt(deg)
    hs_ref[...] = _DOT(x_ref[...], w_ref[...]) * dinv
    dinv_ref[...] = jnp.broadcast_to(dinv, (_R, D))


def _tc_mid_body(p_ref, hsp_ref, dinv_ref, b_ref, w_ref, out_ref):
    agg = p_ref[0] + p_ref[1] + hsp_ref[...]
    f = jnp.maximum(dinv_ref[...] * agg + b_ref[...], 0.0)
    out_ref[...] = _DOT(f, w_ref[...]) * dinv_ref[...]


def _tc_final_body(p_ref, hsp_ref, dinv_ref, b_ref, out_ref):
    agg = p_ref[0] + p_ref[1] + hsp_ref[...]
    out_ref[...] = dinv_ref[...] * agg + b_ref[...]


def _row_spec(width):
    return pl.BlockSpec((_R, width), lambda i: (i, 0))


def _part_spec(width):
    return pl.BlockSpec((NC, _R, width), lambda i: (0, i, 0))


def _full_spec(shape):
    return pl.BlockSpec(shape, lambda i: (0,) * len(shape))


_GRID = N // _R

_tc_prep = pl.pallas_call(
    _tc_prep_body,
    grid=(_GRID,),
    in_specs=[_row_spec(D), _full_spec((D, D)), _part_spec(D)],
    out_specs=[_row_spec(D), _row_spec(D)],
    out_shape=[jax.ShapeDtypeStruct((N, D), jnp.float32),
               jax.ShapeDtypeStruct((N, D), jnp.float32)],
)

_tc_mid = pl.pallas_call(
    _tc_mid_body,
    grid=(_GRID,),
    in_specs=[_part_spec(D), _row_spec(D), _row_spec(D),
              _full_spec((1, D)), _full_spec((D, D))],
    out_specs=_row_spec(D),
    out_shape=jax.ShapeDtypeStruct((N, D), jnp.float32),
)

_tc_final = pl.pallas_call(
    _tc_final_body,
    grid=(_GRID,),
    in_specs=[_part_spec(D), _row_spec(D), _row_spec(D), _full_spec((1, D))],
    out_specs=_row_spec(D),
    out_shape=jax.ShapeDtypeStruct((N, D), jnp.float32),
)


def _pack_edges(idx, fill):
    part0 = idx[:E0].reshape(NS, NCH0, CH)
    tail = jnp.full((CAP1 - (E - E0),), fill, jnp.int32)
    part1 = jnp.concatenate([idx[E0:], tail]).reshape(NS, NCH1, CH)
    part1 = jnp.pad(part1, ((0, 0), (0, NCH0 - NCH1), (0, 0)),
                    constant_values=fill)
    return jnp.concatenate([part0, part1], axis=0)


@jax.jit
def kernel(x, edge_index, W1, b1, W2, b2, W3, b3, W4, b4, W5, b5):
    src3 = _pack_edges(edge_index[0], 0)
    dst3 = _pack_edges(edge_index[1], N)

    zeros_nd = jnp.zeros((NPAD, D), jnp.float32)
    ones_ch = jnp.ones((CH, D), jnp.float32)

    degp = _sc_degree(ones_ch, dst3, zeros_nd)
    hs, dinv = _tc_prep(x, W1, degp)

    W5p = jnp.zeros((D, D), jnp.float32).at[:, :50].set(W5)
    b5p = jnp.zeros((1, D), jnp.float32).at[0, :50].set(b5)

    for b, W in ((b1, W2), (b2, W3), (b3, W4), (b4, W5p)):
        p = _sc_scatter(hs, src3, dst3, zeros_nd)
        hs = _tc_mid(p, hs, dinv, b.reshape(1, D), W)

    p = _sc_scatter(hs, src3, dst3, zeros_nd)
    out = _tc_final(p, hs, dinv, b5p)
    return out[:, :50]
